# baseline (speedup 1.0000x reference)
import functools

import jax
import jax.numpy as jnp
from jax.experimental import pallas as pl
from jax.experimental.pallas import tpu as pltpu


def _gcn_kernel(adj_ref, sim_ref, feat_ref, w1_ref, w2_ref, out_ref,
                af_ref, s2_ref, *, bt, np_):
    w1 = w1_ref[...].astype(jnp.bfloat16)
    w2 = w2_ref[...].astype(jnp.bfloat16)
    n = adj_ref.shape[1]
    abs_ = []
    for g in range(bt):
        a = (adj_ref[g] * sim_ref[g]).astype(jnp.bfloat16)
        abs_.append(a)
        af = jnp.dot(a, feat_ref[g].astype(jnp.bfloat16),
                     preferred_element_type=jnp.float32)
        af_ref[g * np_:g * np_ + n, :] = af.astype(jnp.bfloat16)
    g1 = jnp.maximum(
        jnp.dot(af_ref[...], w1, preferred_element_type=jnp.float32), 0.0)
    s2_ref[...] = jnp.dot(g1.astype(jnp.bfloat16), w2,
                          preferred_element_type=jnp.float32).astype(jnp.bfloat16)
    nh = out_ref.shape[1]
    for g in range(bt):
        s2 = s2_ref[g * np_:g * np_ + n, :]
        g2 = jnp.maximum(
            jnp.dot(abs_[g], s2, preferred_element_type=jnp.float32), 0.0)
        g2p = jnp.pad(g2.astype(jnp.bfloat16), ((0, 2 * nh - n), (0, 0)))
        out_ref[g] = jnp.concatenate([g2p[:nh], g2p[nh:]], axis=1)


def _head_kernel(x_ref, w1_ref, b1_ref, w2_ref, b2_ref, w3_ref, b3_ref,
                 out_ref):
    h1 = jnp.maximum(
        jnp.dot(x_ref[...], w1_ref[...],
                preferred_element_type=jnp.float32) + b1_ref[...], 0.0)
    h2 = jnp.maximum(
        jnp.dot(h1.astype(jnp.bfloat16), w2_ref[...].astype(jnp.bfloat16),
                preferred_element_type=jnp.float32) + b2_ref[...], 0.0)
    out_ref[...] = (
        jnp.dot(h2.astype(jnp.bfloat16), w3_ref[...].astype(jnp.bfloat16),
                preferred_element_type=jnp.float32) + b3_ref[...])


def kernel(adjacency, input_feature, similarity,
           gc1_w, gc2_w, fc1_w, fc1_b, fc2_w, fc2_b, fc3_w, fc3_b):
    B, N, _ = adjacency.shape
    Cin = input_feature.shape[2]
    H = gc2_w.shape[1]
    F1, F2, C = fc1_w.shape[1], fc2_w.shape[1], fc3_w.shape[1]

    bt = 32
    np_ = (N + 7) // 8 * 8
    gcn_flops = B * (N * N + 2 * N * N * Cin + 2 * N * Cin * H
                     + 2 * N * H * H + 2 * N * N * H)
    gcn_bytes = 4 * B * 2 * N * N + 2 * B * N * Cin + 2 * B * N * H \
        + 2 * (Cin * H + H * H)
    np2 = (N + 15) // 16 * 16
    nh = np2 // 2
    hp = 2 * H
    gcn2 = pl.pallas_call(
        functools.partial(_gcn_kernel, bt=bt, np_=np_),
        out_shape=jax.ShapeDtypeStruct((B, nh, hp), jnp.bfloat16),
        grid=(B // bt,),
        in_specs=[
            pl.BlockSpec((bt, N, N), lambda b: (b, 0, 0)),
            pl.BlockSpec((bt, N, N), lambda b: (b, 0, 0)),
            pl.BlockSpec((bt, N, Cin), lambda b: (b, 0, 0)),
            pl.BlockSpec((Cin, H), lambda b: (0, 0)),
            pl.BlockSpec((H, H), lambda b: (0, 0)),
        ],
        out_specs=pl.BlockSpec((bt, nh, hp), lambda b: (b, 0, 0)),
        scratch_shapes=[
            pltpu.VMEM((bt * np_, Cin), jnp.bfloat16),
            pltpu.VMEM((bt * np_, H), jnp.bfloat16),
        ],
        compiler_params=pltpu.CompilerParams(
            dimension_semantics=("parallel",)),
        cost_estimate=pl.CostEstimate(flops=gcn_flops, transcendentals=0,
                                      bytes_accessed=gcn_bytes),
    )(adjacency, similarity, input_feature, gc1_w, gc2_w)

    x = gcn2.reshape(B, nh * hp)
    w3 = fc1_w.reshape(N, H, F1).astype(jnp.bfloat16)
    w3p = jnp.pad(w3, ((0, np2 - N), (0, 0), (0, 0)))
    fc1_wb = jnp.concatenate([w3p[:nh], w3p[nh:]], axis=1).reshape(nh * hp, F1)

    bm = 256 if B % 256 == 0 else B
    head_flops = 2 * B * (N * H * F1 + F1 * F2 + F2 * C)
    head_bytes = 2 * B * nh * hp + 2 * nh * hp * F1 + B * C * 4
    return pl.pallas_call(
        _head_kernel,
        out_shape=jax.ShapeDtypeStruct((B, C), jnp.float32),
        grid=(B // bm,),
        in_specs=[
            pl.BlockSpec((bm, nh * hp), lambda i: (i, 0)),
            pl.BlockSpec((nh * hp, F1), lambda i: (0, 0)),
            pl.BlockSpec((1, F1), lambda i: (0, 0)),
            pl.BlockSpec((F1, F2), lambda i: (0, 0)),
            pl.BlockSpec((1, F2), lambda i: (0, 0)),
            pl.BlockSpec((F2, C), lambda i: (0, 0)),
            pl.BlockSpec((1, C), lambda i: (0, 0)),
        ],
        out_specs=pl.BlockSpec((bm, C), lambda i: (i, 0)),
        compiler_params=pltpu.CompilerParams(
            dimension_semantics=("parallel",)),
        cost_estimate=pl.CostEstimate(flops=head_flops, transcendentals=0,
                                      bytes_accessed=head_bytes),
    )(x, fc1_wb, fc1_b, fc2_w, fc2_b, fc3_w, fc3_b)


# speedup vs baseline: 1.0356x; 1.0356x over previous
import functools

import jax
import jax.numpy as jnp
from jax.experimental import pallas as pl
from jax.experimental.pallas import tpu as pltpu


def _gcn_kernel(adj_ref, sim_ref, feat_ref, w1_ref, w2_ref, out_ref,
                af_ref, s2_ref, *, bt, np_):
    w1 = w1_ref[...].astype(jnp.bfloat16)
    w2 = w2_ref[...].astype(jnp.bfloat16)
    n = adj_ref.shape[1]
    abs_ = []
    for g in range(bt):
        a = (adj_ref[g] * sim_ref[g]).astype(jnp.bfloat16)
        abs_.append(a)
        af = jnp.dot(a, feat_ref[g], preferred_element_type=jnp.float32)
        af_ref[g * np_:g * np_ + n, :] = af.astype(jnp.bfloat16)
    g1 = jnp.maximum(
        jnp.dot(af_ref[...], w1, preferred_element_type=jnp.float32), 0.0)
    s2_ref[...] = jnp.dot(g1.astype(jnp.bfloat16), w2,
                          preferred_element_type=jnp.float32).astype(jnp.bfloat16)
    nh = out_ref.shape[1]
    for g in range(bt):
        s2 = s2_ref[g * np_:g * np_ + n, :]
        g2 = jnp.maximum(
            jnp.dot(abs_[g], s2, preferred_element_type=jnp.float32), 0.0)
        g2p = jnp.pad(g2.astype(jnp.bfloat16), ((0, 2 * nh - n), (0, 0)))
        out_ref[g] = jnp.concatenate([g2p[:nh], g2p[nh:]], axis=1)


def _head_kernel(x_ref, w1_ref, b1_ref, w2_ref, b2_ref, w3_ref, b3_ref,
                 out_ref):
    h1 = jnp.maximum(
        jnp.dot(x_ref[...], w1_ref[...],
                preferred_element_type=jnp.float32) + b1_ref[...], 0.0)
    h2 = jnp.maximum(
        jnp.dot(h1.astype(jnp.bfloat16), w2_ref[...].astype(jnp.bfloat16),
                preferred_element_type=jnp.float32) + b2_ref[...], 0.0)
    out_ref[...] = (
        jnp.dot(h2.astype(jnp.bfloat16), w3_ref[...].astype(jnp.bfloat16),
                preferred_element_type=jnp.float32) + b3_ref[...])


def kernel(adjacency, input_feature, similarity,
           gc1_w, gc2_w, fc1_w, fc1_b, fc2_w, fc2_b, fc3_w, fc3_b):
    B, N, _ = adjacency.shape
    Cin = input_feature.shape[2]
    H = gc2_w.shape[1]
    F1, F2, C = fc1_w.shape[1], fc2_w.shape[1], fc3_w.shape[1]

    feat_b = input_feature.astype(jnp.bfloat16)

    bt = 32
    np_ = (N + 7) // 8 * 8
    gcn_flops = B * (N * N + 2 * N * N * Cin + 2 * N * Cin * H
                     + 2 * N * H * H + 2 * N * N * H)
    gcn_bytes = 4 * B * 2 * N * N + 2 * B * N * Cin + 2 * B * N * H \
        + 2 * (Cin * H + H * H)
    np2 = (N + 15) // 16 * 16
    nh = np2 // 2
    hp = 2 * H
    gcn2 = pl.pallas_call(
        functools.partial(_gcn_kernel, bt=bt, np_=np_),
        out_shape=jax.ShapeDtypeStruct((B, nh, hp), jnp.bfloat16),
        grid=(B // bt,),
        in_specs=[
            pl.BlockSpec((bt, N, N), lambda b: (b, 0, 0)),
            pl.BlockSpec((bt, N, N), lambda b: (b, 0, 0)),
            pl.BlockSpec((bt, N, Cin), lambda b: (b, 0, 0)),
            pl.BlockSpec((Cin, H), lambda b: (0, 0)),
            pl.BlockSpec((H, H), lambda b: (0, 0)),
        ],
        out_specs=pl.BlockSpec((bt, nh, hp), lambda b: (b, 0, 0)),
        scratch_shapes=[
            pltpu.VMEM((bt * np_, Cin), jnp.bfloat16),
            pltpu.VMEM((bt * np_, H), jnp.bfloat16),
        ],
        compiler_params=pltpu.CompilerParams(
            dimension_semantics=("parallel",)),
        cost_estimate=pl.CostEstimate(flops=gcn_flops, transcendentals=0,
                                      bytes_accessed=gcn_bytes),
    )(adjacency, similarity, feat_b, gc1_w, gc2_w)

    x = gcn2.reshape(B, nh * hp)
    w3 = fc1_w.reshape(N, H, F1).astype(jnp.bfloat16)
    w3p = jnp.pad(w3, ((0, np2 - N), (0, 0), (0, 0)))
    fc1_wb = jnp.concatenate([w3p[:nh], w3p[nh:]], axis=1).reshape(nh * hp, F1)

    bm = 128 if B % 128 == 0 else B
    head_flops = 2 * B * (N * H * F1 + F1 * F2 + F2 * C)
    head_bytes = 2 * B * nh * hp + 2 * nh * hp * F1 + B * C * 4
    return pl.pallas_call(
        _head_kernel,
        out_shape=jax.ShapeDtypeStruct((B, C), jnp.float32),
        grid=(B // bm,),
        in_specs=[
            pl.BlockSpec((bm, nh * hp), lambda i: (i, 0)),
            pl.BlockSpec((nh * hp, F1), lambda i: (0, 0)),
            pl.BlockSpec((1, F1), lambda i: (0, 0)),
            pl.BlockSpec((F1, F2), lambda i: (0, 0)),
            pl.BlockSpec((1, F2), lambda i: (0, 0)),
            pl.BlockSpec((F2, C), lambda i: (0, 0)),
            pl.BlockSpec((1, C), lambda i: (0, 0)),
        ],
        out_specs=pl.BlockSpec((bm, C), lambda i: (i, 0)),
        compiler_params=pltpu.CompilerParams(
            dimension_semantics=("parallel",)),
        cost_estimate=pl.CostEstimate(flops=head_flops, transcendentals=0,
                                      bytes_accessed=head_bytes),
    )(x, fc1_wb, fc1_b, fc2_w, fc2_b, fc3_w, fc3_b)
